# Initial kernel scaffold; baseline (speedup 1.0000x reference)
import jax
import jax.numpy as jnp
from jax.experimental import pallas as pl


def kernel(edge_attr, W0, W1, W2):
    raise NotImplementedError("write your pallas kernel here")


# speedup vs baseline: 1.1793x; 1.1793x over previous
import functools

import jax
import jax.numpy as jnp
from jax import lax
from jax.experimental import pallas as pl
from jax.experimental.pallas import tpu as pltpu
from jax.experimental.pallas import tpu_sc as plsc

E = 800000
D = 64
L = 16
NC = 2
NS = 16
NW = NC * NS
TOTAL_G = E // L
G_LO = TOTAL_G // NW
EXTRA = TOTAL_G - G_LO * NW
CG = 64

D0, D1, D2 = 5, 6, 2


def _body(ea_hbm, w0_hbm, w1_hbm, w2_hbm, out_hbm,
          w0_v, w1_v, w2_v, tab_v, ea_v, out_v):
    cid = lax.axis_index("c")
    sid = lax.axis_index("s")
    wid = sid * NC + cid

    pltpu.sync_copy(w0_hbm, w0_v)
    pltpu.sync_copy(w1_hbm, w1_v)
    pltpu.sync_copy(w2_hbm, w2_v)

    for i0 in range(D0):
        for i1 in range(D1):
            s01 = [w0_v[pl.ds(i0 * D + cg * L, L)]
                   + w1_v[pl.ds(i1 * D + cg * L, L)]
                   for cg in range(D // L)]
            for i2 in range(D2):
                base = ((i0 * D1 + i1) * D2 + i2) * D
                for cg in range(D // L):
                    tab_v[pl.ds(base + cg * L, L)] = (
                        s01[cg] + w2_v[pl.ds(i2 * D + cg * L, L)])

    ng = G_LO + (wid < EXTRA).astype(jnp.int32)
    g0 = wid * G_LO + jnp.minimum(wid, EXTRA)
    nfull = ng // CG

    iota = lax.iota(jnp.int32, L)
    i3 = iota * 3
    o64 = iota * D

    def do_chunk(gstart):
        pltpu.sync_copy(ea_hbm.at[pl.ds(gstart * (3 * L), CG * 3 * L)], ea_v)

        def grp(j, carry):
            idx = i3 + j * (3 * L)
            e0 = plsc.load_gather(ea_v, [idx])
            e1 = plsc.load_gather(ea_v, [idx + 1])
            e2 = plsc.load_gather(ea_v, [idx + 2])
            comb = e0 * (D1 * D2) + e1 * D2 + e2
            rb = comb * D
            ob = o64 + j * (L * D)
            for col in range(D):
                v = plsc.load_gather(tab_v, [rb + col])
                plsc.store_scatter(out_v, [ob + col], v)
            return carry

        lax.fori_loop(0, CG, grp, 0)
        pltpu.sync_copy(out_v, out_hbm.at[pl.ds(gstart * (L * D), CG * L * D)])

    def chunk_loop(i, carry):
        do_chunk(g0 + i * CG)
        return carry

    lax.fori_loop(0, nfull, chunk_loop, 0)
    do_chunk(g0 + ng - CG)


_sc_call = functools.partial(
    pl.kernel,
    out_type=jax.ShapeDtypeStruct((E * D,), jnp.float32),
    mesh=plsc.VectorSubcoreMesh(core_axis_name="c", subcore_axis_name="s"),
    compiler_params=pltpu.CompilerParams(needs_layout_passes=False),
    scratch_types=[
        pltpu.VMEM((D0 * D,), jnp.float32),
        pltpu.VMEM((D1 * D,), jnp.float32),
        pltpu.VMEM((D2 * D,), jnp.float32),
        pltpu.VMEM((D0 * D1 * D2 * D,), jnp.float32),
        pltpu.VMEM((CG * 3 * L,), jnp.int32),
        pltpu.VMEM((CG * L * D,), jnp.float32),
    ],
)(_body)


@jax.jit
def kernel(edge_attr, W0, W1, W2):
    ea = edge_attr.reshape(-1).astype(jnp.int32)
    out = _sc_call(ea, W0.reshape(-1), W1.reshape(-1), W2.reshape(-1))
    return out.reshape(E, D)


# speedup vs baseline: 1.3559x; 1.1497x over previous
import functools

import jax
import jax.numpy as jnp
from jax import lax
from jax.experimental import pallas as pl
from jax.experimental.pallas import tpu as pltpu
from jax.experimental.pallas import tpu_sc as plsc

E = 800000
D = 64
L = 16
NC = 2
NS = 16
NW = NC * NS
TOTAL_G = E // L
G_LO = TOTAL_G // NW
EXTRA = TOTAL_G - G_LO * NW
CG = 56
NFULL = G_LO // CG
BATCH = 8

D0, D1, D2 = 5, 6, 2


def _body(ea_hbm, w0_hbm, w1_hbm, w2_hbm, out_hbm,
          w0_v, w1_v, w2_v, tab_v, ea0_v, ea1_v, out0_v, out1_v,
          os0, os1):
    cid = lax.axis_index("c")
    sid = lax.axis_index("s")
    wid = sid * NC + cid

    pltpu.sync_copy(w0_hbm, w0_v)
    pltpu.sync_copy(w1_hbm, w1_v)
    pltpu.sync_copy(w2_hbm, w2_v)

    for i0 in range(D0):
        for i1 in range(D1):
            s01 = [w0_v[pl.ds(i0 * D + cg * L, L)]
                   + w1_v[pl.ds(i1 * D + cg * L, L)]
                   for cg in range(D // L)]
            for i2 in range(D2):
                base = ((i0 * D1 + i1) * D2 + i2) * D
                for cg in range(D // L):
                    tab_v[pl.ds(base + cg * L, L)] = (
                        s01[cg] + w2_v[pl.ds(i2 * D + cg * L, L)])

    ng = G_LO + (wid < EXTRA).astype(jnp.int32)
    g0 = wid * G_LO + jnp.minimum(wid, EXTRA)

    iota = lax.iota(jnp.int32, L)
    i3 = iota * 3
    o64 = iota * D

    ea_bufs = (ea0_v, ea1_v)
    out_bufs = (out0_v, out1_v)
    out_sems = (os0, os1)

    def compute_chunk(gstart, ea_v, out_v):
        pltpu.sync_copy(ea_hbm.at[pl.ds(gstart * (3 * L), CG * 3 * L)], ea_v)

        def grp(j, carry):
            idx3 = i3 + j * (3 * L)
            e0 = plsc.load_gather(ea_v, [idx3])
            e1 = plsc.load_gather(ea_v, [idx3 + 1])
            e2 = plsc.load_gather(ea_v, [idx3 + 2])
            rb = e0 * (D1 * D2 * D) + e1 * (D2 * D) + e2 * D
            ob = o64 + j * (L * D)
            for cb in range(0, D, BATCH):
                vs = [plsc.load_gather(tab_v, [rb + (cb + k)])
                      for k in range(BATCH)]
                for k in range(BATCH):
                    plsc.store_scatter(out_v, [ob + (cb + k)], vs[k])
            return carry

        lax.fori_loop(0, CG, grp, 0)

    def chunk(i, b):
        gstart = g0 + i * CG

        @pl.when(i >= 2)
        def _():
            pltpu.make_async_copy(
                out_bufs[b],
                out_hbm.at[pl.ds(gstart * (L * D), CG * L * D)],
                out_sems[b]).wait()

        compute_chunk(gstart, ea_bufs[b], out_bufs[b])
        pltpu.make_async_copy(
            out_bufs[b],
            out_hbm.at[pl.ds(gstart * (L * D), CG * L * D)],
            out_sems[b]).start()

    def pair(jp, carry):
        chunk(jp * 2, 0)
        chunk(jp * 2 + 1, 1)
        return carry

    lax.fori_loop(0, NFULL // 2, pair, 0)
    if NFULL % 2:
        chunk(NFULL - 1, 0)

    gtail = g0 + ng - CG

    @pl.when(NFULL >= 2)
    def _():
        pltpu.make_async_copy(
            out_bufs[1],
            out_hbm.at[pl.ds(gtail * (L * D), CG * L * D)],
            out_sems[1]).wait()

    compute_chunk(gtail, ea_bufs[1], out_bufs[1])
    pltpu.sync_copy(out_bufs[1], out_hbm.at[pl.ds(gtail * (L * D), CG * L * D)])
    pltpu.make_async_copy(
        out_bufs[0],
        out_hbm.at[pl.ds(gtail * (L * D), CG * L * D)],
        out_sems[0]).wait()


_sc_call = functools.partial(
    pl.kernel,
    out_type=jax.ShapeDtypeStruct((E * D,), jnp.float32),
    mesh=plsc.VectorSubcoreMesh(core_axis_name="c", subcore_axis_name="s"),
    compiler_params=pltpu.CompilerParams(needs_layout_passes=False),
    scratch_types=[
        pltpu.VMEM((D0 * D,), jnp.float32),
        pltpu.VMEM((D1 * D,), jnp.float32),
        pltpu.VMEM((D2 * D,), jnp.float32),
        pltpu.VMEM((D0 * D1 * D2 * D,), jnp.float32),
        pltpu.VMEM((CG * 3 * L,), jnp.int32),
        pltpu.VMEM((CG * 3 * L,), jnp.int32),
        pltpu.VMEM((CG * L * D,), jnp.float32),
        pltpu.VMEM((CG * L * D,), jnp.float32),
        pltpu.SemaphoreType.DMA,
        pltpu.SemaphoreType.DMA,
    ],
)(_body)


@jax.jit
def kernel(edge_attr, W0, W1, W2):
    ea = edge_attr.reshape(-1).astype(jnp.int32)
    out = _sc_call(ea, W0.reshape(-1), W1.reshape(-1), W2.reshape(-1))
    return out.reshape(E, D)


# speedup vs baseline: 1.3566x; 1.0005x over previous
import functools

import jax
import jax.numpy as jnp
from jax import lax
from jax.experimental import pallas as pl
from jax.experimental.pallas import tpu as pltpu
from jax.experimental.pallas import tpu_sc as plsc

E = 800000
D = 64
L = 16
NC = 2
NS = 16
NW = NC * NS
TOTAL_G = E // L
G_LO = TOTAL_G // NW
EXTRA = TOTAL_G - G_LO * NW
CG = 56
NFULL = G_LO // CG
BATCH = 8
ROWS = CG * L * D // 128

D0, D1, D2 = 5, 6, 2


def _body(ea_hbm, w0_hbm, w1_hbm, w2_hbm, out_hbm,
          w0_v, w1_v, w2_v, tab_v, ea0_v, ea1_v, out0_v, out1_v,
          os0, os1):
    cid = lax.axis_index("c")
    sid = lax.axis_index("s")
    wid = sid * NC + cid

    pltpu.sync_copy(w0_hbm, w0_v)
    pltpu.sync_copy(w1_hbm, w1_v)
    pltpu.sync_copy(w2_hbm, w2_v)

    for i0 in range(D0):
        for i1 in range(D1):
            s01 = [w0_v[pl.ds(i0 * D + cg * L, L)]
                   + w1_v[pl.ds(i1 * D + cg * L, L)]
                   for cg in range(D // L)]
            for i2 in range(D2):
                base = ((i0 * D1 + i1) * D2 + i2) * D
                for cg in range(D // L):
                    tab_v[pl.ds(base + cg * L, L)] = (
                        s01[cg] + w2_v[pl.ds(i2 * D + cg * L, L)])

    ng = G_LO + (wid < EXTRA).astype(jnp.int32)
    g0 = wid * G_LO + jnp.minimum(wid, EXTRA)

    ea_bufs = (ea0_v, ea1_v)
    out_bufs = (out0_v, out1_v)
    out_sems = (os0, os1)

    iota = lax.iota(jnp.int32, L)
    i3 = iota * 3
    rhalf = lax.shift_right_logical(iota, 1)
    cbase = (iota & 1) * D

    def compute_chunk(gstart, ea_v, out_v):
        pltpu.sync_copy(ea_hbm.at[pl.ds(gstart * (3 * L), CG * 3 * L)], ea_v)

        def grp(j, carry):
            idx3 = i3 + j * (3 * L)
            e0 = plsc.load_gather(ea_v, [idx3])
            e1 = plsc.load_gather(ea_v, [idx3 + 1])
            e2 = plsc.load_gather(ea_v, [idx3 + 2])
            rb = e0 * (D1 * D2 * D) + e1 * (D2 * D) + e2 * D
            rvec = rhalf + j * 8
            for cb in range(0, D, BATCH):
                vs = [plsc.load_gather(tab_v, [rb + (cb + k)])
                      for k in range(BATCH)]
                for k in range(BATCH):
                    plsc.store_scatter(out_v, [rvec, cbase + (cb + k)],
                                       vs[k])
            return carry

        lax.fori_loop(0, CG, grp, 0)

    def chunk(i, b):
        gstart = g0 + i * CG

        @pl.when(i >= 2)
        def _():
            pltpu.make_async_copy(
                out_bufs[b],
                out_hbm.at[pl.ds(gstart * 8, ROWS)],
                out_sems[b]).wait()

        compute_chunk(gstart, ea_bufs[b], out_bufs[b])
        pltpu.make_async_copy(
            out_bufs[b],
            out_hbm.at[pl.ds(gstart * 8, ROWS)],
            out_sems[b]).start()

    def pair(jp, carry):
        chunk(jp * 2, 0)
        chunk(jp * 2 + 1, 1)
        return carry

    lax.fori_loop(0, NFULL // 2, pair, 0)
    if NFULL % 2:
        chunk(NFULL - 1, 0)

    gtail = g0 + ng - CG

    @pl.when(NFULL >= 2)
    def _():
        pltpu.make_async_copy(
            out_bufs[1],
            out_hbm.at[pl.ds(gtail * 8, ROWS)],
            out_sems[1]).wait()

    compute_chunk(gtail, ea_bufs[1], out_bufs[1])
    pltpu.sync_copy(out_bufs[1], out_hbm.at[pl.ds(gtail * 8, ROWS)])
    pltpu.make_async_copy(
        out_bufs[0],
        out_hbm.at[pl.ds(gtail * 8, ROWS)],
        out_sems[0]).wait()


_sc_call = functools.partial(
    pl.kernel,
    out_type=jax.ShapeDtypeStruct((E * D // 128, 128), jnp.float32),
    mesh=plsc.VectorSubcoreMesh(core_axis_name="c", subcore_axis_name="s"),
    compiler_params=pltpu.CompilerParams(needs_layout_passes=False),
    scratch_types=[
        pltpu.VMEM((D0 * D,), jnp.float32),
        pltpu.VMEM((D1 * D,), jnp.float32),
        pltpu.VMEM((D2 * D,), jnp.float32),
        pltpu.VMEM((D0 * D1 * D2 * D,), jnp.float32),
        pltpu.VMEM((CG * 3 * L,), jnp.int32),
        pltpu.VMEM((CG * 3 * L,), jnp.int32),
        pltpu.VMEM((ROWS, 128), jnp.float32),
        pltpu.VMEM((ROWS, 128), jnp.float32),
        pltpu.SemaphoreType.DMA,
        pltpu.SemaphoreType.DMA,
    ],
)(_body)


@jax.jit
def kernel(edge_attr, W0, W1, W2):
    ea = edge_attr.reshape(-1).astype(jnp.int32)
    out = _sc_call(ea, W0.reshape(-1), W1.reshape(-1), W2.reshape(-1))
    return out.reshape(E, D)


# speedup vs baseline: 12.5320x; 9.2376x over previous
import functools

import jax
import jax.numpy as jnp
from jax import lax
from jax.experimental import pallas as pl
from jax.experimental.pallas import tpu as pltpu
from jax.experimental.pallas import tpu_sc as plsc

E = 800000
D = 64
L = 16
NC = 2
NS = 16
NW = NC * NS
EB = E // 128
TOTAL_G = E // L
B_LO = EB // NW
EXTRA_B = EB - B_LO * NW
CG = 48
NFULL = (B_LO * 8) // CG
CEB = CG // 8

D0, D1, D2 = 5, 6, 2
TROWS = 64


def _body(ea_hbm, w0_hbm, w1_hbm, w2_hbm, out_hbm,
          w0_v, w1_v, w2_v, tab_v, tabt_v, ea0_v, ea1_v, out0_v, out1_v,
          os0, os1):
    cid = lax.axis_index("c")
    sid = lax.axis_index("s")
    wid = sid * NC + cid

    pltpu.sync_copy(w0_hbm, w0_v)
    pltpu.sync_copy(w1_hbm, w1_v)
    pltpu.sync_copy(w2_hbm, w2_v)

    for i0 in range(D0):
        for i1 in range(D1):
            s01 = [w0_v[pl.ds(i0 * D + cg * L, L)]
                   + w1_v[pl.ds(i1 * D + cg * L, L)]
                   for cg in range(D // L)]
            for i2 in range(D2):
                base = ((i0 * D1 + i1) * D2 + i2) * D
                for cg in range(D // L):
                    tab_v[pl.ds(base + cg * L, L)] = (
                        s01[cg] + w2_v[pl.ds(i2 * D + cg * L, L)])

    iota = lax.iota(jnp.int32, L)

    for c in range(D):
        for rg in range(TROWS // L):
            v = plsc.load_gather(tab_v, [(iota + rg * L) * D + c])
            tabt_v[pl.ds(c * TROWS + rg * L, L)] = v

    ng = (B_LO + (wid < EXTRA_B).astype(jnp.int32)) * 8
    g0 = (wid * B_LO + jnp.minimum(wid, EXTRA_B)) * 8

    ea_bufs = (ea0_v, ea1_v)
    out_bufs = (out0_v, out1_v)
    out_sems = (os0, os1)

    CE = CG * L

    def compute_chunk(gstart, ea_v, out_v):
        estart = gstart * L
        pltpu.sync_copy(ea_hbm.at[pl.ds(estart, CE)], ea_v.at[pl.ds(0, CE)])
        pltpu.sync_copy(ea_hbm.at[pl.ds(E + estart, CE)],
                        ea_v.at[pl.ds(CE, CE)])
        pltpu.sync_copy(ea_hbm.at[pl.ds(2 * E + estart, CE)],
                        ea_v.at[pl.ds(2 * CE, CE)])

        def grp(j, carry):
            e0 = ea_v[pl.ds(j * L, L)]
            e1 = ea_v[pl.ds(CE + j * L, L)]
            e2 = ea_v[pl.ds(2 * CE + j * L, L)]
            comb = e0 * (D1 * D2) + e1 * D2 + e2
            ebl = lax.shift_right_logical(j, 3)
            elb = (j & 7) * L
            for cb in range(8):
                for ci in range(8):
                    c = cb * 8 + ci
                    v = plsc.load_gather(tabt_v, [comb + c * TROWS])
                    out_v[cb, ebl, ci, pl.ds(elb, L)] = v
            return carry

        lax.fori_loop(0, CG, grp, 0)

    def chunk(i, b):
        gstart = g0 + i * CG
        eb0 = lax.shift_right_logical(gstart, 3)

        @pl.when(i >= 2)
        def _():
            pltpu.make_async_copy(
                out_bufs[b],
                out_hbm.at[:, pl.ds(eb0, CEB)],
                out_sems[b]).wait()

        compute_chunk(gstart, ea_bufs[b], out_bufs[b])
        pltpu.make_async_copy(
            out_bufs[b],
            out_hbm.at[:, pl.ds(eb0, CEB)],
            out_sems[b]).start()

    def pair(jp, carry):
        chunk(jp * 2, 0)
        chunk(jp * 2 + 1, 1)
        return carry

    lax.fori_loop(0, NFULL // 2, pair, 0)
    if NFULL % 2:
        chunk(NFULL - 1, 0)

    gtail = g0 + ng - CG
    ebt = lax.shift_right_logical(gtail, 3)

    @pl.when(NFULL >= 2)
    def _():
        pltpu.make_async_copy(
            out_bufs[1],
            out_hbm.at[:, pl.ds(ebt, CEB)],
            out_sems[1]).wait()

    compute_chunk(gtail, ea_bufs[1], out_bufs[1])
    pltpu.sync_copy(out_bufs[1], out_hbm.at[:, pl.ds(ebt, CEB)])
    pltpu.make_async_copy(
        out_bufs[0],
        out_hbm.at[:, pl.ds(ebt, CEB)],
        out_sems[0]).wait()


_sc_call = functools.partial(
    pl.kernel,
    out_type=jax.ShapeDtypeStruct((8, EB, 8, 128), jnp.float32),
    mesh=plsc.VectorSubcoreMesh(core_axis_name="c", subcore_axis_name="s"),
    compiler_params=pltpu.CompilerParams(needs_layout_passes=False),
    scratch_types=[
        pltpu.VMEM((D0 * D,), jnp.float32),
        pltpu.VMEM((D1 * D,), jnp.float32),
        pltpu.VMEM((D2 * D,), jnp.float32),
        pltpu.VMEM((TROWS * D,), jnp.float32),
        pltpu.VMEM((D * TROWS,), jnp.float32),
        pltpu.VMEM((CG * 3 * L,), jnp.int32),
        pltpu.VMEM((CG * 3 * L,), jnp.int32),
        pltpu.VMEM((8, CEB, 8, 128), jnp.float32),
        pltpu.VMEM((8, CEB, 8, 128), jnp.float32),
        pltpu.SemaphoreType.DMA,
        pltpu.SemaphoreType.DMA,
    ],
)(_body)


@jax.jit
def kernel(edge_attr, W0, W1, W2):
    ea = edge_attr.T.reshape(-1).astype(jnp.int32)
    buf = _sc_call(ea, W0.reshape(-1), W1.reshape(-1), W2.reshape(-1))
    return buf.transpose(1, 3, 0, 2).reshape(E, D)


# speedup vs baseline: 24.7701x; 1.9765x over previous
import functools

import jax
import jax.numpy as jnp
from jax import lax
from jax.experimental import pallas as pl
from jax.experimental.pallas import tpu as pltpu
from jax.experimental.pallas import tpu_sc as plsc

E = 800000
D = 64
L = 16
NC = 2
NS = 16
NW = NC * NS
EB = E // 128
TOTAL_G = E // L
B_LO = EB // NW
EXTRA_B = EB - B_LO * NW
CG = 48
NFULL = (B_LO * 8) // CG
CEB = CG // 8

D0, D1, D2 = 5, 6, 2
TROWS = 64
BATCH = 8


def _body(ea_hbm, w0_hbm, w1_hbm, w2_hbm, out_hbm,
          w0_v, w1_v, w2_v, tab_v, tabt_v, ea0_v, ea1_v, out0_v, out1_v,
          os0, os1):
    cid = lax.axis_index("c")
    sid = lax.axis_index("s")
    wid = sid * NC + cid

    pltpu.sync_copy(w0_hbm, w0_v)
    pltpu.sync_copy(w1_hbm, w1_v)
    pltpu.sync_copy(w2_hbm, w2_v)

    for i0 in range(D0):
        for i1 in range(D1):
            s01 = [w0_v[pl.ds(i0 * D + cg * L, L)]
                   + w1_v[pl.ds(i1 * D + cg * L, L)]
                   for cg in range(D // L)]
            for i2 in range(D2):
                base = ((i0 * D1 + i1) * D2 + i2) * D
                for cg in range(D // L):
                    tab_v[pl.ds(base + cg * L, L)] = (
                        s01[cg] + w2_v[pl.ds(i2 * D + cg * L, L)])

    iota = lax.iota(jnp.int32, L)

    for c in range(D):
        for rg in range(TROWS // L):
            v = plsc.load_gather(tab_v, [(iota + rg * L) * D + c])
            tabt_v[pl.ds(c * TROWS + rg * L, L)] = v

    ng = (B_LO + (wid < EXTRA_B).astype(jnp.int32)) * 8
    g0 = (wid * B_LO + jnp.minimum(wid, EXTRA_B)) * 8

    ea_bufs = (ea0_v, ea1_v)
    out_bufs = (out0_v, out1_v)
    out_sems = (os0, os1)

    CE = CG * L

    def compute_chunk(gstart, ea_v, out_v):
        estart = gstart * L
        pltpu.sync_copy(ea_hbm.at[pl.ds(estart, CE)], ea_v.at[pl.ds(0, CE)])
        pltpu.sync_copy(ea_hbm.at[pl.ds(E + estart, CE)],
                        ea_v.at[pl.ds(CE, CE)])
        pltpu.sync_copy(ea_hbm.at[pl.ds(2 * E + estart, CE)],
                        ea_v.at[pl.ds(2 * CE, CE)])

        def grp(j, carry):
            e0 = ea_v[pl.ds(j * L, L)]
            e1 = ea_v[pl.ds(CE + j * L, L)]
            e2 = ea_v[pl.ds(2 * CE + j * L, L)]
            comb = e0 * (D1 * D2) + e1 * D2 + e2
            ebl = lax.shift_right_logical(j, 3)
            elb = (j & 7) * L
            for c0 in range(0, D, BATCH):
                vs = [plsc.load_gather(tabt_v, [comb + (c0 + k) * TROWS])
                      for k in range(BATCH)]
                for k in range(BATCH):
                    c = c0 + k
                    out_v[c // 8, ebl, c % 8, pl.ds(elb, L)] = vs[k]
            return carry

        lax.fori_loop(0, CG, grp, 0)

    def chunk(i, b):
        gstart = g0 + i * CG
        eb0 = lax.shift_right_logical(gstart, 3)

        @pl.when(i >= 2)
        def _():
            pltpu.make_async_copy(
                out_bufs[b],
                out_hbm.at[:, pl.ds(eb0, CEB)],
                out_sems[b]).wait()

        compute_chunk(gstart, ea_bufs[b], out_bufs[b])
        pltpu.make_async_copy(
            out_bufs[b],
            out_hbm.at[:, pl.ds(eb0, CEB)],
            out_sems[b]).start()

    def pair(jp, carry):
        chunk(jp * 2, 0)
        chunk(jp * 2 + 1, 1)
        return carry

    lax.fori_loop(0, NFULL // 2, pair, 0)
    if NFULL % 2:
        chunk(NFULL - 1, 0)

    gtail = g0 + ng - CG
    ebt = lax.shift_right_logical(gtail, 3)

    @pl.when(NFULL >= 2)
    def _():
        pltpu.make_async_copy(
            out_bufs[1],
            out_hbm.at[:, pl.ds(ebt, CEB)],
            out_sems[1]).wait()

    compute_chunk(gtail, ea_bufs[1], out_bufs[1])
    pltpu.sync_copy(out_bufs[1], out_hbm.at[:, pl.ds(ebt, CEB)])
    pltpu.make_async_copy(
        out_bufs[0],
        out_hbm.at[:, pl.ds(ebt, CEB)],
        out_sems[0]).wait()


_sc_call = functools.partial(
    pl.kernel,
    out_type=jax.ShapeDtypeStruct((8, EB, 8, 128), jnp.float32),
    mesh=plsc.VectorSubcoreMesh(core_axis_name="c", subcore_axis_name="s"),
    compiler_params=pltpu.CompilerParams(needs_layout_passes=False),
    scratch_types=[
        pltpu.VMEM((D0 * D,), jnp.float32),
        pltpu.VMEM((D1 * D,), jnp.float32),
        pltpu.VMEM((D2 * D,), jnp.float32),
        pltpu.VMEM((TROWS * D,), jnp.float32),
        pltpu.VMEM((D * TROWS,), jnp.float32),
        pltpu.VMEM((CG * 3 * L,), jnp.int32),
        pltpu.VMEM((CG * 3 * L,), jnp.int32),
        pltpu.VMEM((8, CEB, 8, 128), jnp.float32),
        pltpu.VMEM((8, CEB, 8, 128), jnp.float32),
        pltpu.SemaphoreType.DMA,
        pltpu.SemaphoreType.DMA,
    ],
)(_body)


@jax.jit
def kernel(edge_attr, W0, W1, W2):
    ea = edge_attr.T.reshape(-1).astype(jnp.int32)
    buf = _sc_call(ea, W0.reshape(-1), W1.reshape(-1), W2.reshape(-1))
    return buf.transpose(1, 3, 0, 2).reshape(E, D)


# speedup vs baseline: 35.6790x; 1.4404x over previous
import functools

import jax
import jax.numpy as jnp
from jax import lax
from jax.experimental import pallas as pl
from jax.experimental.pallas import tpu as pltpu
from jax.experimental.pallas import tpu_sc as plsc

E = 800000
D = 64
L = 16
NC = 2
NS = 16
NW = NC * NS
EB = E // 128
TOTAL_G = E // L
B_LO = EB // NW
EXTRA_B = EB - B_LO * NW
CG = 48
NFULL = (B_LO * 8) // CG
CEB = CG // 8

D0, D1, D2 = 5, 6, 2
TROWS = 64
BATCH = 8


def _body(ea_hbm, w0_hbm, w1_hbm, w2_hbm, out_hbm,
          w0_v, w1_v, w2_v, tab_v, tabt_v, ea0_v, ea1_v, out0_v, out1_v,
          os0, os1, es0, es1):
    cid = lax.axis_index("c")
    sid = lax.axis_index("s")
    wid = sid * NC + cid

    pltpu.sync_copy(w0_hbm, w0_v)
    pltpu.sync_copy(w1_hbm, w1_v)
    pltpu.sync_copy(w2_hbm, w2_v)

    for i0 in range(D0):
        for i1 in range(D1):
            s01 = [w0_v[pl.ds(i0 * D + cg * L, L)]
                   + w1_v[pl.ds(i1 * D + cg * L, L)]
                   for cg in range(D // L)]
            for i2 in range(D2):
                base = ((i0 * D1 + i1) * D2 + i2) * D
                for cg in range(D // L):
                    tab_v[pl.ds(base + cg * L, L)] = (
                        s01[cg] + w2_v[pl.ds(i2 * D + cg * L, L)])

    iota = lax.iota(jnp.int32, L)

    for c in range(D):
        for rg in range(TROWS // L):
            v = plsc.load_gather(tab_v, [(iota + rg * L) * D + c])
            tabt_v[pl.ds(c * TROWS + rg * L, L)] = v

    ng = (B_LO + (wid < EXTRA_B).astype(jnp.int32)) * 8
    g0 = (wid * B_LO + jnp.minimum(wid, EXTRA_B)) * 8

    ea_bufs = (ea0_v, ea1_v)
    out_bufs = (out0_v, out1_v)
    out_sems = (os0, os1)
    ea_sems = (es0, es1)

    CE = CG * L
    gtail = g0 + ng - CG

    def start_ea(gstart, b):
        estart = gstart * L
        for f in range(3):
            pltpu.make_async_copy(
                ea_hbm.at[pl.ds(f * E + estart, CE)],
                ea_bufs[b].at[pl.ds(f * CE, CE)],
                ea_sems[b]).start()

    def wait_ea(b):
        for f in range(3):
            pltpu.make_async_copy(
                ea_hbm.at[pl.ds(0, CE)],
                ea_bufs[b].at[pl.ds(f * CE, CE)],
                ea_sems[b]).wait()

    def compute_chunk(ea_v, out_v):

        def grp(j, carry):
            e0 = ea_v[pl.ds(j * L, L)]
            e1 = ea_v[pl.ds(CE + j * L, L)]
            e2 = ea_v[pl.ds(2 * CE + j * L, L)]
            comb = e0 * (D1 * D2) + e1 * D2 + e2
            ebl = lax.shift_right_logical(j, 3)
            elb = (j & 7) * L
            for c0 in range(0, D, BATCH):
                vs = [plsc.load_gather(tabt_v, [comb + (c0 + k) * TROWS])
                      for k in range(BATCH)]
                for k in range(BATCH):
                    c = c0 + k
                    out_v[c // 8, ebl, c % 8, pl.ds(elb, L)] = vs[k]
            return carry

        lax.fori_loop(0, CG, grp, 0)

    start_ea(g0, 0)
    start_ea(g0 + CG, 1)

    def chunk(i, b):
        gstart = g0 + i * CG
        eb0 = lax.shift_right_logical(gstart, 3)

        @pl.when(i >= 2)
        def _():
            pltpu.make_async_copy(
                out_bufs[b],
                out_hbm.at[:, pl.ds(eb0, CEB)],
                out_sems[b]).wait()

        wait_ea(b)
        compute_chunk(ea_bufs[b], out_bufs[b])
        pltpu.make_async_copy(
            out_bufs[b],
            out_hbm.at[:, pl.ds(eb0, CEB)],
            out_sems[b]).start()
        start_ea(jnp.minimum(gstart + 2 * CG, gtail), b)

    def pair(jp, carry):
        chunk(jp * 2, 0)
        chunk(jp * 2 + 1, 1)
        return carry

    lax.fori_loop(0, NFULL // 2, pair, 0)
    if NFULL % 2:
        chunk(NFULL - 1, 0)

    ebt = lax.shift_right_logical(gtail, 3)

    pltpu.make_async_copy(
        out_bufs[1],
        out_hbm.at[:, pl.ds(ebt, CEB)],
        out_sems[1]).wait()
    wait_ea(1)
    compute_chunk(ea_bufs[1], out_bufs[1])
    pltpu.sync_copy(out_bufs[1], out_hbm.at[:, pl.ds(ebt, CEB)])
    pltpu.make_async_copy(
        out_bufs[0],
        out_hbm.at[:, pl.ds(ebt, CEB)],
        out_sems[0]).wait()
    wait_ea(0)


_sc_call = functools.partial(
    pl.kernel,
    out_type=jax.ShapeDtypeStruct((8, EB, 8, 128), jnp.float32),
    mesh=plsc.VectorSubcoreMesh(core_axis_name="c", subcore_axis_name="s"),
    compiler_params=pltpu.CompilerParams(needs_layout_passes=False),
    scratch_types=[
        pltpu.VMEM((D0 * D,), jnp.float32),
        pltpu.VMEM((D1 * D,), jnp.float32),
        pltpu.VMEM((D2 * D,), jnp.float32),
        pltpu.VMEM((TROWS * D,), jnp.float32),
        pltpu.VMEM((D * TROWS,), jnp.float32),
        pltpu.VMEM((CG * 3 * L,), jnp.int32),
        pltpu.VMEM((CG * 3 * L,), jnp.int32),
        pltpu.VMEM((8, CEB, 8, 128), jnp.float32),
        pltpu.VMEM((8, CEB, 8, 128), jnp.float32),
        pltpu.SemaphoreType.DMA,
        pltpu.SemaphoreType.DMA,
        pltpu.SemaphoreType.DMA,
        pltpu.SemaphoreType.DMA,
    ],
)(_body)


@jax.jit
def kernel(edge_attr, W0, W1, W2):
    ea = edge_attr.T.reshape(-1).astype(jnp.int32)
    buf = _sc_call(ea, W0.reshape(-1), W1.reshape(-1), W2.reshape(-1))
    return buf.transpose(1, 3, 0, 2).reshape(E, D)
